# Initial kernel scaffold; baseline (speedup 1.0000x reference)
import jax
import jax.numpy as jnp
from jax.experimental import pallas as pl


def kernel(input, indices):
    raise NotImplementedError("write your pallas kernel here")


# speedup vs baseline: 3.1625x; 3.1625x over previous
import functools

import jax
import jax.numpy as jnp
from jax import lax
from jax.experimental import pallas as pl
from jax.experimental.pallas import tpu as pltpu
from jax.experimental.pallas import tpu_sc as plsc

_NC = 2
_NS = 16
_NW = _NC * _NS
_C = 128
_NBUF = 4


@functools.lru_cache(maxsize=None)
def _make_gather(V, D, B):
    rows_per_w = B // _NW
    K = rows_per_w // _C
    G = K // _NBUF
    assert B == _NW * K * _C and K % _NBUF == 0 and G >= 2

    mesh = plsc.VectorSubcoreMesh(core_axis_name="c", subcore_axis_name="s")

    @functools.partial(
        pl.kernel,
        mesh=mesh,
        out_type=jax.ShapeDtypeStruct((B, D), jnp.float32),
        scratch_types=[
            pltpu.VMEM((K, _C), jnp.int32),
            pltpu.VMEM((_NBUF, _C, D), jnp.float32),
        ] + [pltpu.SemaphoreType.DMA] * (2 * _NBUF),
    )
    def gather_k(table, idx, out, idx_v, rows_v, *sems):
        gsems = sems[:_NBUF]
        osems = sems[_NBUF:]
        w = lax.axis_index("s") * _NC + lax.axis_index("c")
        chunk0 = w * K
        pltpu.sync_copy(idx.at[pl.ds(chunk0, K), :], idx_v)

        def do_chunk(g, b, wait_out):
            c = g * _NBUF + b
            orow = (chunk0 + c) * _C
            gcopy = pltpu.make_async_copy(
                table.at[idx_v.at[c]], rows_v.at[b], gsems[b])
            ocopy = pltpu.make_async_copy(
                rows_v.at[b], out.at[pl.ds(orow, _C), :], osems[b])
            if wait_out:
                ocopy.wait()
            gcopy.start()
            gcopy.wait()
            ocopy.start()

        for b in range(_NBUF):
            do_chunk(0, b, False)

        def body(g, carry):
            for b in range(_NBUF):
                do_chunk(g, b, True)
            return carry

        lax.fori_loop(1, G, body, 0)

        for b in range(_NBUF):
            pltpu.make_async_copy(
                rows_v.at[b], out.at[pl.ds(chunk0 * _C, _C), :], osems[b]
            ).wait()

    return gather_k


def kernel(input, indices):
    V, D = input.shape
    B = indices.size
    idx = indices.reshape(-1).astype(jnp.int32).reshape(B // _C, _C)
    out = _make_gather(V, D, B)(input, idx)
    return out.reshape(*indices.shape, D)


# speedup vs baseline: 3.3998x; 1.0750x over previous
import functools

import jax
import jax.numpy as jnp
from jax import lax
from jax.experimental import pallas as pl
from jax.experimental.pallas import tpu as pltpu
from jax.experimental.pallas import tpu_sc as plsc

_NC = 2
_NS = 16
_NW = _NC * _NS
_C = 104
_NBUF = 8
_D = 6


@functools.lru_cache(maxsize=None)
def _make_gather(V, D, B):
    rows_per_w = B // _NW
    K = rows_per_w // _C
    G = K // _NBUF
    assert B == _NW * K * _C and K % _NBUF == 0 and G >= 3 and _D < _NBUF

    mesh = plsc.VectorSubcoreMesh(core_axis_name="c", subcore_axis_name="s")

    @functools.partial(
        pl.kernel,
        mesh=mesh,
        out_type=jax.ShapeDtypeStruct((B, D), jnp.float32),
        scratch_types=[
            pltpu.VMEM((K, _C), jnp.int32),
            pltpu.VMEM((_NBUF, _C, D), jnp.float32),
        ] + [pltpu.SemaphoreType.DMA] * (2 * _NBUF),
    )
    def gather_k(table, idx, out, idx_v, rows_v, *sems):
        gsems = sems[:_NBUF]
        osems = sems[_NBUF:]
        w = lax.axis_index("s") * _NC + lax.axis_index("c")
        chunk0 = w * K
        pltpu.sync_copy(idx.at[pl.ds(chunk0, K), :], idx_v)

        def gcopy(c, b):
            return pltpu.make_async_copy(
                table.at[idx_v.at[c]], rows_v.at[b], gsems[b])

        def ocopy(c, b):
            return pltpu.make_async_copy(
                rows_v.at[b], out.at[pl.ds((chunk0 + c) * _C, _C), :],
                osems[b])

        def step(c, b, wait_o, prefetch):
            gcopy(c, b).wait()
            oc = ocopy(c, b)
            oc.start()
            if prefetch:
                cf = c + _D
                bf = (b + _D) % _NBUF
                if wait_o:
                    ocopy(c, bf).wait()
                gcopy(cf, bf).start()

        for b in range(_D):
            gcopy(b, b).start()

        for b in range(_NBUF):
            step(b, b, wait_o=(b + _D >= _NBUF), prefetch=True)

        def body(g, carry):
            for b in range(_NBUF):
                step(g * _NBUF + b, b, wait_o=True, prefetch=True)
            return carry

        lax.fori_loop(1, G - 1, body, 0)

        for b in range(_NBUF):
            c = (G - 1) * _NBUF + b
            step(c, b, wait_o=True, prefetch=(b + _D < _NBUF))

        for b in range(_NBUF):
            ocopy(0, b).wait()

    return gather_k


def kernel(input, indices):
    V, D = input.shape
    B = indices.size
    idx = indices.reshape(-1).astype(jnp.int32).reshape(B // _C, _C)
    out = _make_gather(V, D, B)(input, idx)
    return out.reshape(*indices.shape, D)


# speedup vs baseline: 5.6642x; 1.6660x over previous
import functools

import jax
import jax.numpy as jnp
from jax import lax
from jax.experimental import pallas as pl
from jax.experimental.pallas import tpu as pltpu
from jax.experimental.pallas import tpu_sc as plsc

_NC = 2
_NS = 16
_NW = _NC * _NS
_SLAB = 26
_SPC = 4
_C = _SLAB * _SPC
_NBUF = 8
_D = 6


@functools.lru_cache(maxsize=None)
def _make_gather(V, D, N):
    B = N * _SLAB
    rows_per_w = B // _NW
    K = rows_per_w // _C
    G = K // _NBUF
    assert B == _NW * K * _C and K % _NBUF == 0 and G >= 3 and _D < _NBUF

    mesh = plsc.VectorSubcoreMesh(core_axis_name="c", subcore_axis_name="s")

    @functools.partial(
        pl.kernel,
        mesh=mesh,
        out_type=jax.ShapeDtypeStruct((N, _SLAB, D), jnp.float32),
        scratch_types=[
            pltpu.VMEM((rows_per_w,), jnp.int32),
            pltpu.VMEM((_NBUF, _C, D), jnp.float32),
        ] + [pltpu.SemaphoreType.DMA] * (2 * _NBUF),
        compiler_params=pltpu.CompilerParams(use_tc_tiling_on_sc=True),
    )
    def gather_k(table, idx, out, idx_v, rows_v, *sems):
        gsems = sems[:_NBUF]
        osems = sems[_NBUF:]
        w = lax.axis_index("s") * _NC + lax.axis_index("c")
        chunk0 = w * K
        pltpu.sync_copy(idx.at[pl.ds(chunk0 * _C, rows_per_w)], idx_v)

        def gcopy(c, b):
            return pltpu.make_async_copy(
                table.at[idx_v.at[pl.ds(c * _C, _C)]], rows_v.at[b],
                gsems[b])

        def ostart(c, b):
            for s in range(_SPC):
                slab = (chunk0 + c) * _SPC + s
                pltpu.make_async_copy(
                    rows_v.at[b, pl.ds(s * _SLAB, _SLAB)], out.at[slab],
                    osems[b]).start()

        def owait(b):
            for s in range(_SPC):
                pltpu.make_async_copy(
                    rows_v.at[b, pl.ds(s * _SLAB, _SLAB)],
                    out.at[chunk0 * _SPC + s], osems[b]).wait()

        def step(c, b, wait_o, prefetch):
            gcopy(c, b).wait()
            ostart(c, b)
            if prefetch:
                cf = c + _D
                bf = (b + _D) % _NBUF
                if wait_o:
                    owait(bf)
                gcopy(cf, bf).start()

        for b in range(_D):
            gcopy(b, b).start()

        for b in range(_NBUF):
            step(b, b, wait_o=(b + _D >= _NBUF), prefetch=True)

        def body(g, carry):
            for b in range(_NBUF):
                step(g * _NBUF + b, b, wait_o=True, prefetch=True)
            return carry

        lax.fori_loop(1, G - 1, body, 0)

        for b in range(_NBUF):
            c = (G - 1) * _NBUF + b
            step(c, b, wait_o=True, prefetch=(b + _D < _NBUF))

        for b in range(_NBUF):
            owait(b)

    return gather_k


def kernel(input, indices):
    V, D = input.shape
    N, S = indices.shape
    assert S == _SLAB
    idx = indices.reshape(-1).astype(jnp.int32)
    return _make_gather(V, D, N)(input, idx)


# speedup vs baseline: 11.8725x; 2.0961x over previous
import functools

import jax
import jax.numpy as jnp
from jax import lax
from jax.experimental import pallas as pl
from jax.experimental.pallas import tpu as pltpu
from jax.experimental.pallas import tpu_sc as plsc

_NC = 2
_NS = 16
_NW = _NC * _NS
_C = 128
_NBUF = 4
_D = 3


@functools.lru_cache(maxsize=None)
def _make_gather(V, D, N, S):
    B = N * S
    rows_per_w = B // _NW
    K = rows_per_w // _C
    G = K // _NBUF
    cpj = N // _C
    assert B == _NW * K * _C and K % _NBUF == 0 and G >= 3 and _D < _NBUF
    assert N % _C == 0 and cpj == 128

    mesh = plsc.VectorSubcoreMesh(core_axis_name="c", subcore_axis_name="s")

    @functools.partial(
        pl.kernel,
        mesh=mesh,
        out_type=jax.ShapeDtypeStruct((S, N, D), jnp.float32),
        scratch_types=[
            pltpu.VMEM((rows_per_w,), jnp.int32),
            pltpu.VMEM((_NBUF, _C, D), jnp.float32),
        ] + [pltpu.SemaphoreType.DMA] * (2 * _NBUF),
    )
    def gather_k(table, idx, out, idx_v, rows_v, *sems):
        gsems = sems[:_NBUF]
        osems = sems[_NBUF:]
        w = lax.axis_index("s") * _NC + lax.axis_index("c")
        chunk0 = w * K
        pltpu.sync_copy(idx.at[pl.ds(chunk0 * _C, rows_per_w)], idx_v)

        def gcopy(c, b):
            return pltpu.make_async_copy(
                table.at[idx_v.at[pl.ds(c * _C, _C)]], rows_v.at[b],
                gsems[b])

        def ocopy(c, b):
            cg = chunk0 + c
            j = cg // cpj
            i0 = (cg % cpj) * _C
            return pltpu.make_async_copy(
                rows_v.at[b], out.at[j, pl.ds(i0, _C), :], osems[b])

        def step(c, b, wait_o, prefetch):
            gcopy(c, b).wait()
            ocopy(c, b).start()
            if prefetch:
                cf = c + _D
                bf = (b + _D) % _NBUF
                if wait_o:
                    ocopy(c, bf).wait()
                gcopy(cf, bf).start()

        for b in range(_D):
            gcopy(b, b).start()

        for b in range(_NBUF):
            step(b, b, wait_o=(b + _D >= _NBUF), prefetch=True)

        def body(g, carry):
            for b in range(_NBUF):
                step(g * _NBUF + b, b, wait_o=True, prefetch=True)
            return carry

        lax.fori_loop(1, G - 1, body, 0)

        for b in range(_NBUF):
            c = (G - 1) * _NBUF + b
            step(c, b, wait_o=True, prefetch=(b + _D < _NBUF))

        for b in range(_NBUF):
            ocopy(0, b).wait()

    return gather_k


def kernel(input, indices):
    V, D = input.shape
    N, S = indices.shape
    idx = indices.T.reshape(-1).astype(jnp.int32)
    out = _make_gather(V, D, N, S)(input, idx)
    return out.transpose(1, 0, 2)
